# Initial kernel scaffold; baseline (speedup 1.0000x reference)
import functools

import jax
import jax.numpy as jnp
from jax import lax
from jax.experimental import pallas as pl
from jax.experimental.pallas import tpu as pltpu

_AN2 = 25
_C = 32
_K = 3
_PS = 4
_H = 32
_W = 32
_PN = _H // _PS
_NP = _PN * _PN
_CU = _C * _PS * _PS
_BIG = jnp.float32(3.4e38)


def _match_kernel(x_ref, wt_ref, y_ref):
    x = x_ref[0]
    xx = x * x
    ones_r = jnp.ones((1, _CU), dtype=jnp.float32)
    g = lax.dot_general(x, x, (((1,), (1,)), ((), ())),
                        preferred_element_type=jnp.float32)
    ncol = lax.dot_general(xx, ones_r, (((1,), (1,)), ((), ())),
                           preferred_element_type=jnp.float32)
    nrow = lax.dot_general(ones_r, xx, (((1,), (1,)), ((), ())),
                           preferred_element_type=jnp.float32)
    d = ncol + nrow - 2.0 * g
    pad = jnp.broadcast_to(ncol, (_NP, _NP))
    cur = jnp.concatenate([d, pad], axis=1)
    col = lax.broadcasted_iota(jnp.int32, (_NP, 2 * _NP), 1)
    col64 = lax.broadcasted_iota(jnp.int32, (_NP, _NP), 1)

    acc = jnp.zeros((_NP, _CU), dtype=jnp.float32)
    for k in range(_K):
        m = jnp.min(cur, axis=1, keepdims=True)
        idx = jnp.min(jnp.where(cur == m, col, 2 * _NP),
                      axis=1, keepdims=True)
        cur = jnp.where(col == idx, _BIG, cur)
        onehot = ((col64 == idx) & (idx < _NP)).astype(jnp.float32)
        sel = lax.dot_general(onehot, x, (((1,), (0,)), ((), ())),
                              preferred_element_type=jnp.float32)
        acc = acc + lax.dot_general(sel, wt_ref[k],
                                    (((1,), (0,)), ((), ())),
                                    preferred_element_type=jnp.float32)
    y_ref[0] = jnp.where(acc >= 0, acc, 0.1 * acc)


def _conv3_kernel(z_ref, w2_ref, o_ref):
    wi = lax.broadcasted_iota(jnp.int32, (2 * _C, _H * _W), 1) % _W
    acc = jnp.zeros((_C, _H * _W), dtype=jnp.float32)
    for dy in range(3):
        zs = z_ref[0, :, dy:dy + _H, :].reshape(2 * _C, _H * _W)
        for dx in range(3):
            s = dx - 1
            if s == 0:
                zshift = zs
            else:
                zshift = pltpu.roll(zs, -s, 1)
                if s == 1:
                    mask = wi < (_W - 1)
                else:
                    mask = wi >= 1
                zshift = jnp.where(mask, zshift, 0.0)
            acc = acc + lax.dot_general(
                w2_ref[dy * 3 + dx], zshift, (((1,), (0,)), ((), ())),
                preferred_element_type=jnp.float32)
    o_ref[0] = jnp.where(acc >= 0, acc, 0.1 * acc)


@jax.jit
def kernel(lf_fea, w_agg1, w_agg2):
    B = lf_fea.shape[0]
    x = lf_fea.reshape(B, _C, _PN, _PS, _PN, _PS)
    x = x.transpose(0, 2, 4, 1, 3, 5).reshape(B, _NP, _CU)
    w1r = w_agg1.reshape(_C, _K, _C)
    eye = jnp.eye(_PS * _PS, dtype=jnp.float32)
    wt = jnp.stack([jnp.kron(w1r[:, k, :].T, eye) for k in range(_K)])

    y_patch = pl.pallas_call(
        _match_kernel,
        grid=(B,),
        in_specs=[
            pl.BlockSpec((1, _NP, _CU), lambda b: (b, 0, 0)),
            pl.BlockSpec((_K, _CU, _CU), lambda b: (0, 0, 0)),
        ],
        out_specs=pl.BlockSpec((1, _NP, _CU), lambda b: (b, 0, 0)),
        out_shape=jax.ShapeDtypeStruct((B, _NP, _CU), jnp.float32),
    )(x, wt)

    y_img = y_patch.reshape(B, _PN, _PN, _C, _PS, _PS)
    y_img = y_img.transpose(0, 3, 1, 4, 2, 5).reshape(B, _C, _H, _W)
    z = jnp.concatenate([lf_fea, y_img], axis=1)
    z = jnp.pad(z, ((0, 0), (0, 0), (1, 1), (0, 0)))
    w2r = w_agg2.transpose(2, 3, 0, 1).reshape(9, _C, 2 * _C)

    out = pl.pallas_call(
        _conv3_kernel,
        grid=(B,),
        in_specs=[
            pl.BlockSpec((1, 2 * _C, _H + 2, _W), lambda b: (b, 0, 0, 0)),
            pl.BlockSpec((9, _C, 2 * _C), lambda b: (0, 0, 0)),
        ],
        out_specs=pl.BlockSpec((1, _C, _H * _W), lambda b: (b, 0, 0)),
        out_shape=jax.ShapeDtypeStruct((B, _C, _H * _W), jnp.float32),
    )(z, w2r)
    return out.reshape(B, _C, _H, _W)


# speedup vs baseline: 27.2648x; 27.2648x over previous
import functools

import jax
import jax.numpy as jnp
from jax import lax
from jax.experimental import pallas as pl
from jax.experimental.pallas import tpu as pltpu

_AN2 = 25
_C = 32
_K = 3
_PS = 4
_H = 32
_W = 32
_PN = _H // _PS
_NP = _PN * _PN
_CU = _C * _PS * _PS
_BIG = 3.4e38


def _match_kernel(x_ref, wt_ref, y_ref):
    x = x_ref[0]
    xx = x * x
    ones_r = jnp.ones((1, _CU), dtype=jnp.float32)
    g = lax.dot_general(x, x, (((1,), (1,)), ((), ())),
                        preferred_element_type=jnp.float32)
    ncol = lax.dot_general(xx, ones_r, (((1,), (1,)), ((), ())),
                           preferred_element_type=jnp.float32)
    nrow = lax.dot_general(ones_r, xx, (((1,), (1,)), ((), ())),
                           preferred_element_type=jnp.float32)
    d = ncol + nrow - 2.0 * g
    pad = jnp.broadcast_to(ncol, (_NP, _NP))
    cur = jnp.concatenate([d, pad], axis=1)
    col = lax.broadcasted_iota(jnp.int32, (_NP, 2 * _NP), 1)
    col64 = lax.broadcasted_iota(jnp.int32, (_NP, _NP), 1)

    acc = jnp.zeros((_NP, _CU), dtype=jnp.float32)
    for k in range(_K):
        m = jnp.min(cur, axis=1, keepdims=True)
        idx = jnp.min(jnp.where(cur == m, col, 2 * _NP),
                      axis=1, keepdims=True)
        cur = jnp.where(col == idx, _BIG, cur)
        onehot = ((col64 == idx) & (idx < _NP)).astype(jnp.float32)
        sel = lax.dot_general(onehot, x, (((1,), (0,)), ((), ())),
                              preferred_element_type=jnp.float32)
        acc = acc + lax.dot_general(sel, wt_ref[k],
                                    (((1,), (0,)), ((), ())),
                                    preferred_element_type=jnp.float32)
    y_ref[0] = jnp.where(acc >= 0, acc, 0.1 * acc)


def _conv3_kernel(z_ref, w2_ref, o_ref):
    wi = lax.broadcasted_iota(jnp.int32, (2 * _C, _H * _W), 1) % _W
    acc = jnp.zeros((_C, _H * _W), dtype=jnp.float32)
    for dy in range(3):
        zs = z_ref[0, :, dy:dy + _H, :].reshape(2 * _C, _H * _W)
        for dx in range(3):
            s = dx - 1
            if s == 0:
                zshift = zs
            else:
                zshift = pltpu.roll(zs, (-s) % (_H * _W), 1)
                if s == 1:
                    mask = wi < (_W - 1)
                else:
                    mask = wi >= 1
                zshift = jnp.where(mask, zshift, 0.0)
            acc = acc + lax.dot_general(
                w2_ref[dy * 3 + dx], zshift, (((1,), (0,)), ((), ())),
                preferred_element_type=jnp.float32)
    o_ref[0] = jnp.where(acc >= 0, acc, 0.1 * acc)


@jax.jit
def kernel(lf_fea, w_agg1, w_agg2):
    B = lf_fea.shape[0]
    x = lf_fea.reshape(B, _C, _PN, _PS, _PN, _PS)
    x = x.transpose(0, 2, 4, 1, 3, 5).reshape(B, _NP, _CU)
    w1r = w_agg1.reshape(_C, _K, _C)
    eye = jnp.eye(_PS * _PS, dtype=jnp.float32)
    wt = jnp.stack([jnp.kron(w1r[:, k, :].T, eye) for k in range(_K)])

    y_patch = pl.pallas_call(
        _match_kernel,
        grid=(B,),
        in_specs=[
            pl.BlockSpec((1, _NP, _CU), lambda b: (b, 0, 0)),
            pl.BlockSpec((_K, _CU, _CU), lambda b: (0, 0, 0)),
        ],
        out_specs=pl.BlockSpec((1, _NP, _CU), lambda b: (b, 0, 0)),
        out_shape=jax.ShapeDtypeStruct((B, _NP, _CU), jnp.float32),
    )(x, wt)

    y_img = y_patch.reshape(B, _PN, _PN, _C, _PS, _PS)
    y_img = y_img.transpose(0, 3, 1, 4, 2, 5).reshape(B, _C, _H, _W)
    z = jnp.concatenate([lf_fea, y_img], axis=1)
    z = jnp.pad(z, ((0, 0), (0, 0), (1, 1), (0, 0)))
    w2r = w_agg2.transpose(2, 3, 0, 1).reshape(9, _C, 2 * _C)

    out = pl.pallas_call(
        _conv3_kernel,
        grid=(B,),
        in_specs=[
            pl.BlockSpec((1, 2 * _C, _H + 2, _W), lambda b: (b, 0, 0, 0)),
            pl.BlockSpec((9, _C, 2 * _C), lambda b: (0, 0, 0)),
        ],
        out_specs=pl.BlockSpec((1, _C, _H * _W), lambda b: (b, 0, 0)),
        out_shape=jax.ShapeDtypeStruct((B, _C, _H * _W), jnp.float32),
    )(z, w2r)
    return out.reshape(B, _C, _H, _W)


# speedup vs baseline: 52.3781x; 1.9211x over previous
import numpy as np

import jax
import jax.numpy as jnp
from jax import lax
from jax.experimental import pallas as pl
from jax.experimental.pallas import tpu as pltpu

_C = 32
_K = 3
_PS = 4
_H = 32
_W = 32
_PN = _H // _PS
_NP = _PN * _PN
_PP = _PS * _PS
_PIX = _H * _W
_BIG = 3.4e38


def _extraction_matrices():
    e = np.zeros((_PIX, _PP * _NP), dtype=np.float32)
    for psh in range(_PS):
        for psw in range(_PS):
            pp = psh * _PS + psw
            for hr in range(_PN):
                for wr in range(_PN):
                    q = hr * _PN + wr
                    pix = (_PS * hr + psh) * _W + _PS * wr + psw
                    e[pix, pp * _NP + q] = 1.0
    e2 = e.reshape(_PIX, _PP, _NP).transpose(1, 2, 0).copy()
    return e, e2


_E_NP, _E2_NP = _extraction_matrices()


def _leaky(v):
    return jnp.where(v >= 0, v, 0.1 * v)


def _match_kernel(lf_ref, e_ref, w1_ref, yp_ref):
    lf = lf_ref[0]
    a = lax.dot_general(lf, e_ref[...], (((1,), (0,)), ((), ())),
                        preferred_element_type=jnp.float32)
    aa = a * a
    ones_r = jnp.ones((1, _C), dtype=jnp.float32)
    g = jnp.zeros((_NP, _NP), dtype=jnp.float32)
    ncol = jnp.zeros((_NP, 1), dtype=jnp.float32)
    for pp in range(_PP):
        ap = a[:, pp * _NP:(pp + 1) * _NP]
        aap = aa[:, pp * _NP:(pp + 1) * _NP]
        g = g + lax.dot_general(ap, ap, (((0,), (0,)), ((), ())),
                                preferred_element_type=jnp.float32)
        ncol = ncol + lax.dot_general(aap, ones_r, (((0,), (1,)), ((), ())),
                                      preferred_element_type=jnp.float32)
    s = lax.dot_general(ones_r, aa, (((1,), (0,)), ((), ())),
                        preferred_element_type=jnp.float32)
    nrow = jnp.zeros((1, _NP), dtype=jnp.float32)
    for pp in range(_PP):
        nrow = nrow + s[:, pp * _NP:(pp + 1) * _NP]
    d = ncol + nrow - 2.0 * g
    pad = jnp.broadcast_to(ncol, (_NP, _NP))
    cur = jnp.concatenate([d, pad], axis=1)
    col = lax.broadcasted_iota(jnp.int32, (_NP, 2 * _NP), 1)
    col64 = lax.broadcasted_iota(jnp.int32, (_NP, _NP), 1)

    sels = []
    for k in range(_K):
        m = jnp.min(cur, axis=1, keepdims=True)
        idx = jnp.min(jnp.where(cur == m, col, 2 * _NP),
                      axis=1, keepdims=True)
        cur = jnp.where(col == idx, _BIG, cur)
        sels.append(((col64 == idx) & (idx < _NP)).astype(jnp.float32))

    for pp in range(_PP):
        ap = a[:, pp * _NP:(pp + 1) * _NP]
        acc = jnp.zeros((_NP, _C), dtype=jnp.float32)
        for k in range(_K):
            z = lax.dot_general(ap, w1_ref[k], (((0,), (1,)), ((), ())),
                                preferred_element_type=jnp.float32)
            acc = acc + lax.dot_general(sels[k], z, (((1,), (0,)), ((), ())),
                                        preferred_element_type=jnp.float32)
        yp_ref[0, :, pp * _C:(pp + 1) * _C] = acc


def _conv_kernel(lf_ref, yp_ref, e2_ref, w2a_ref, w2b_ref, o_ref):
    lf = lf_ref[0]
    y = _leaky(yp_ref[0])
    yimg = jnp.zeros((_C, _PIX), dtype=jnp.float32)
    for pp in range(_PP):
        ys = y[:, pp * _C:(pp + 1) * _C]
        yimg = yimg + lax.dot_general(ys, e2_ref[pp],
                                      (((0,), (0,)), ((), ())),
                                      preferred_element_type=jnp.float32)
    hi = lax.broadcasted_iota(jnp.int32, (_C, _PIX), 1) // _W
    wi = lax.broadcasted_iota(jnp.int32, (_C, _PIX), 1) % _W
    acc = jnp.zeros((_C, _PIX), dtype=jnp.float32)
    for dy in range(3):
        for dx in range(3):
            t = dy * 3 + dx
            s = _W * (dy - 1) + (dx - 1)
            mask = None
            if dy == 0:
                mask = hi >= 1
            elif dy == 2:
                mask = hi < (_H - 1)
            if dx == 0:
                mw = wi >= 1
                mask = mw if mask is None else (mask & mw)
            elif dx == 2:
                mw = wi < (_W - 1)
                mask = mw if mask is None else (mask & mw)
            if s == 0:
                lfs, ys = lf, yimg
            else:
                lfs = pltpu.roll(lf, (-s) % _PIX, 1)
                ys = pltpu.roll(yimg, (-s) % _PIX, 1)
            if mask is not None:
                lfs = jnp.where(mask, lfs, 0.0)
                ys = jnp.where(mask, ys, 0.0)
            acc = acc + lax.dot_general(w2a_ref[t], lfs,
                                        (((1,), (0,)), ((), ())),
                                        preferred_element_type=jnp.float32)
            acc = acc + lax.dot_general(w2b_ref[t], ys,
                                        (((1,), (0,)), ((), ())),
                                        preferred_element_type=jnp.float32)
    o_ref[0] = _leaky(acc)


@jax.jit
def kernel(lf_fea, w_agg1, w_agg2):
    B = lf_fea.shape[0]
    lf2d = lf_fea.reshape(B, _C, _PIX)
    e = jnp.asarray(_E_NP)
    e2 = jnp.asarray(_E2_NP)
    w1 = w_agg1.reshape(_C, _K, _C).transpose(1, 0, 2)
    w2a = w_agg2[:, :_C].transpose(2, 3, 0, 1).reshape(9, _C, _C)
    w2b = w_agg2[:, _C:].transpose(2, 3, 0, 1).reshape(9, _C, _C)

    yp = pl.pallas_call(
        _match_kernel,
        grid=(B,),
        in_specs=[
            pl.BlockSpec((1, _C, _PIX), lambda b: (b, 0, 0)),
            pl.BlockSpec((_PIX, _PIX), lambda b: (0, 0)),
            pl.BlockSpec((_K, _C, _C), lambda b: (0, 0, 0)),
        ],
        out_specs=pl.BlockSpec((1, _NP, _PP * _C), lambda b: (b, 0, 0)),
        out_shape=jax.ShapeDtypeStruct((B, _NP, _PP * _C), jnp.float32),
    )(lf2d, e, w1)

    out = pl.pallas_call(
        _conv_kernel,
        grid=(B,),
        in_specs=[
            pl.BlockSpec((1, _C, _PIX), lambda b: (b, 0, 0)),
            pl.BlockSpec((1, _NP, _PP * _C), lambda b: (b, 0, 0)),
            pl.BlockSpec((_PP, _NP, _PIX), lambda b: (0, 0, 0)),
            pl.BlockSpec((9, _C, _C), lambda b: (0, 0, 0)),
            pl.BlockSpec((9, _C, _C), lambda b: (0, 0, 0)),
        ],
        out_specs=pl.BlockSpec((1, _C, _PIX), lambda b: (b, 0, 0)),
        out_shape=jax.ShapeDtypeStruct((B, _C, _PIX), jnp.float32),
    )(lf2d, yp, e2, w2a, w2b)
    return out.reshape(B, _C, _H, _W)
